# baseline (speedup 1.0000x reference)
import numpy as np
import jax
import jax.numpy as jnp
from jax.experimental import pallas as pl

_ALPHA = 4


def _slow_idx(t: int) -> list:
    n = t // _ALPHA
    return [int(v) for v in np.linspace(0.0, t - 1, n).astype(np.int32)]


def kernel(frames):
    c, t, h, w = frames.shape
    idx = _slow_idx(t)
    n = len(idx)
    hw = h * w

    rows = frames.reshape(c * t, 1, hw)

    def body(in_ref, fast_ref, slow_ref):
        r = pl.program_id(0)
        tt = jax.lax.rem(r, t)
        x = in_ref[...]
        fast_ref[...] = x
        sel = tt == idx[0]
        for k in idx[1:]:
            sel = jnp.logical_or(sel, tt == k)

        @pl.when(sel)
        def _():
            slow_ref[...] = x

    def slow_map(r):
        tt = jax.lax.rem(r, t)
        ch = jax.lax.div(r, t)
        inv = jnp.int32(0)
        for k in idx:
            inv = inv + jnp.where(tt > k, jnp.int32(1), jnp.int32(0))
        return (ch * n + inv, 0, 0)

    slow, fast = pl.pallas_call(
        body,
        grid=(c * t,),
        in_specs=[pl.BlockSpec((1, 1, hw), lambda r: (r, 0, 0))],
        out_specs=[
            pl.BlockSpec((1, 1, hw), slow_map),
            pl.BlockSpec((1, 1, hw), lambda r: (r, 0, 0)),
        ],
        out_shape=[
            jax.ShapeDtypeStruct((c * n, 1, hw), frames.dtype),
            jax.ShapeDtypeStruct((c * t, 1, hw), frames.dtype),
        ],
    )(rows)
    return (slow.reshape(c, n, h, w), fast.reshape(c, t, h, w))


# speedup vs baseline: 3.1990x; 3.1990x over previous
import numpy as np
import jax
import jax.numpy as jnp
from jax.experimental import pallas as pl

_ALPHA = 4


def _slow_idx(t: int) -> list:
    n = t // _ALPHA
    return [int(v) for v in np.linspace(0.0, t - 1, n).astype(np.int32)]


def kernel(frames):
    c, t, h, w = frames.shape
    idx = _slow_idx(t)
    n = len(idx)
    hw = h * w
    g_sz = t // n
    offs = [idx[g] - g_sz * g for g in range(n)]
    assert all(0 <= o < g_sz for o in offs)

    groups = frames.reshape(c * n, g_sz, hw)

    def body(in_ref, slow_ref, fast_ref):
        j = pl.program_id(0)
        g = jax.lax.rem(j, n)
        x = in_ref[...]
        fast_ref[...] = x
        off = jnp.int32(offs[0])
        for k in range(1, n):
            off = jnp.where(g == k, jnp.int32(offs[k]), off)
        slow_ref[...] = in_ref[:, pl.ds(off, 1), :]

    slow, fast = pl.pallas_call(
        body,
        grid=(c * n,),
        in_specs=[pl.BlockSpec((1, g_sz, hw), lambda j: (j, 0, 0))],
        out_specs=[
            pl.BlockSpec((1, 1, hw), lambda j: (j, 0, 0)),
            pl.BlockSpec((1, g_sz, hw), lambda j: (j, 0, 0)),
        ],
        out_shape=[
            jax.ShapeDtypeStruct((c * n, 1, hw), frames.dtype),
            jax.ShapeDtypeStruct((c * n, g_sz, hw), frames.dtype),
        ],
    )(groups)
    return (slow.reshape(c, n, h, w), fast.reshape(c, t, h, w))
